# baseline (speedup 1.0000x reference)
import jax
import jax.numpy as jnp
from jax import lax
from jax.experimental import pallas as pl
from jax.experimental.pallas import tpu as pltpu
from jax.experimental.pallas import tpu_sc as plsc

NUM_CORES = 2
NUM_SUBCORES = 16
NW = NUM_CORES * NUM_SUBCORES

NUM_ROWS = 1000000
BATCH = 16384
EMBED_K = 64
BPW = BATCH // NW


def _sc_body(users_h, items_h, gut_h, git_h, xui_h, gut_o, git_o,
             idx_u, idx_i, panel_u, panel_i, xui_v,
             sem_u, sem_i, sem_ou, sem_oi):
    wid = lax.axis_index("s") * NUM_CORES + lax.axis_index("c")
    base = wid * BPW

    pltpu.sync_copy(users_h.at[pl.ds(base, BPW)], idx_u)
    pltpu.sync_copy(items_h.at[pl.ds(base, BPW)], idx_i)

    def gather_k(k, _):
        cu = pltpu.async_copy(gut_h.at[k].at[idx_u], panel_u.at[k], sem_u)
        ci = pltpu.async_copy(git_h.at[k].at[idx_i], panel_i.at[k], sem_i)
        cu.wait()
        ci.wait()
        return _

    lax.fori_loop(0, EMBED_K, gather_k, 0)

    ou = pltpu.async_copy(panel_u, gut_o.at[:, pl.ds(base, BPW)], sem_ou)
    oi = pltpu.async_copy(panel_i, git_o.at[:, pl.ds(base, BPW)], sem_oi)

    def dot_k(k, _):
        def vec(g, c):
            u16 = panel_u[k, pl.ds(g * 16, 16)]
            i16 = panel_i[k, pl.ds(g * 16, 16)]
            xui_v[pl.ds(g * 16, 16)] = xui_v[pl.ds(g * 16, 16)] + u16 * i16
            return c
        lax.fori_loop(0, BPW // 16, vec, 0)
        return _

    def zero(g, c):
        xui_v[pl.ds(g * 16, 16)] = jnp.zeros((16,), jnp.float32)
        return c

    lax.fori_loop(0, BPW // 16, zero, 0)
    lax.fori_loop(0, EMBED_K, dot_k, 0)

    pltpu.sync_copy(xui_v, xui_h.at[pl.ds(base, BPW)])
    ou.wait()
    oi.wait()


def _sc_gather(users, items, GuT, GiT):
    mesh = plsc.VectorSubcoreMesh(
        core_axis_name="c", subcore_axis_name="s",
        num_cores=NUM_CORES, num_subcores=NUM_SUBCORES)
    return pl.kernel(
        _sc_body,
        out_type=(
            jax.ShapeDtypeStruct((BATCH,), jnp.float32),
            jax.ShapeDtypeStruct((EMBED_K, BATCH), jnp.float32),
            jax.ShapeDtypeStruct((EMBED_K, BATCH), jnp.float32),
        ),
        mesh=mesh,
        compiler_params=pltpu.CompilerParams(
            needs_layout_passes=False, use_tc_tiling_on_sc=False),
        scratch_types=[
            pltpu.VMEM((BPW,), jnp.int32),
            pltpu.VMEM((BPW,), jnp.int32),
            pltpu.VMEM((EMBED_K, BPW), jnp.float32),
            pltpu.VMEM((EMBED_K, BPW), jnp.float32),
            pltpu.VMEM((BPW,), jnp.float32),
            pltpu.SemaphoreType.DMA,
            pltpu.SemaphoreType.DMA,
            pltpu.SemaphoreType.DMA,
            pltpu.SemaphoreType.DMA,
        ],
    )(users, items, GuT, GiT)


@jax.jit
def _impl(users, items, Gu, Gi):
    xui, gut, git = _sc_gather(users, items, Gu.T, Gi.T)
    return xui, gut.T, git.T


def kernel(users, items, Gu, Gi):
    xui, gamma_u, gamma_i = _impl(users, items, Gu, Gi)
    return (xui, gamma_u, gamma_i)
